# Initial kernel scaffold; baseline (speedup 1.0000x reference)
import jax
import jax.numpy as jnp
from jax.experimental import pallas as pl

B, T, C, S = 16, 512, 2048, 64
S2 = 2 * S


def _emis_kernel(llh_ref, s2pn_ref, s2pd_ref, out_ref):
    llh = llh_ref[0]
    s2p = jnp.concatenate([s2pn_ref[0], s2pd_ref[...]], axis=-1)
    cidx = jax.lax.broadcasted_iota(jnp.int32, (C, S2), 0)
    onehot = (cidx == s2p).astype(jnp.float32)
    out_ref[0] = jnp.dot(llh, onehot, preferred_element_type=jnp.float32)


def _fwd_kernel(emis_ref, nA_ref, dA_ref, nI_ref, dI_ref, nF_ref, dF_ref,
                seql_ref, out_ref):
    nA = nA_ref[...]
    dA = dA_ref[...]
    seql = seql_ref[...]
    e0 = emis_ref[0]
    an = nI_ref[...] + e0[:, :S]
    ad = jnp.broadcast_to(dI_ref[...], (B, S)) + e0[:, S:]

    def step(t, carry):
        an, ad = carry
        et = emis_ref[t]
        xn = an[:, :, None] + nA
        mn = jnp.max(xn, axis=1)
        cn = mn + jnp.log(jnp.sum(jnp.exp(xn - mn[:, None, :]), axis=1))
        cn = cn + et[:, :S]
        xd = ad[:, :, None] + dA
        md = jnp.max(xd, axis=1)
        cd = md + jnp.log(jnp.sum(jnp.exp(xd - md[:, None, :]), axis=1))
        cd = cd + et[:, S:]
        act = t < seql
        return jnp.where(act, cn, an), jnp.where(act, cd, ad)

    an, ad = jax.lax.fori_loop(1, T, step, (an, ad))

    nf = an + nF_ref[...]
    df = ad + jnp.broadcast_to(dF_ref[...], (B, S))
    mn = jnp.max(nf, axis=1, keepdims=True)
    num = mn + jnp.log(jnp.sum(jnp.exp(nf - mn), axis=1, keepdims=True))
    md = jnp.max(df, axis=1, keepdims=True)
    den = md + jnp.log(jnp.sum(jnp.exp(df - md), axis=1, keepdims=True))
    out_ref[0, 0] = -jnp.sum(num - den)


def _impl(input, seqlengths, num_logA, num_init, num_final, num_state2pdf,
          den_logA, den_init, den_final, den_state2pdf, interpret=False):
    emis = pl.pallas_call(
        _emis_kernel,
        grid=(B,),
        in_specs=[
            pl.BlockSpec((1, T, C), lambda b: (b, 0, 0)),
            pl.BlockSpec((1, 1, S), lambda b: (b, 0, 0)),
            pl.BlockSpec((1, S), lambda b: (0, 0)),
        ],
        out_specs=pl.BlockSpec((1, T, S2), lambda b: (b, 0, 0)),
        out_shape=jax.ShapeDtypeStruct((B, T, S2), jnp.float32),
        interpret=interpret,
    )(input, num_state2pdf.reshape(B, 1, S), den_state2pdf.reshape(1, S))
    emis_t = jnp.transpose(emis, (1, 0, 2))
    loss = pl.pallas_call(
        _fwd_kernel,
        out_shape=jax.ShapeDtypeStruct((1, 1), jnp.float32),
        interpret=interpret,
    )(emis_t, num_logA, den_logA.reshape(1, S, S), num_init,
      den_init.reshape(1, S), num_final, den_final.reshape(1, S),
      seqlengths.reshape(B, 1))
    return loss[0, 0]


def kernel(input, seqlengths, num_logA, num_init, num_final, num_state2pdf,
           den_logA, den_init, den_final, den_state2pdf):
    return _impl(input, seqlengths, num_logA, num_init, num_final,
                 num_state2pdf, den_logA, den_init, den_final, den_state2pdf)


# speedup vs baseline: 10.6812x; 10.6812x over previous
import jax
import jax.numpy as jnp
from jax.experimental import pallas as pl

B, T, C, S = 16, 512, 2048, 64
S2 = 2 * S


def _emis_kernel(llh_ref, s2pn_ref, s2pd_ref, out_ref):
    llh = llh_ref[0]
    s2p = jnp.concatenate([s2pn_ref[0], s2pd_ref[...]], axis=-1)
    cidx = jax.lax.broadcasted_iota(jnp.int32, (C, S2), 0)
    onehot = (cidx == s2p).astype(jnp.float32)
    out_ref[0] = jnp.dot(llh, onehot, preferred_element_type=jnp.float32)


def _fwd_kernel(emis_ref, nA_ref, dA_ref, nI_ref, dI_ref, nF_ref, dF_ref,
                seql_ref, out_ref):
    nA = nA_ref[...]
    dA = dA_ref[...]
    seql = seql_ref[...]
    e0 = emis_ref[0]
    an = nI_ref[...] + e0[:, :S]
    ad = jnp.broadcast_to(dI_ref[...], (B, S)) + e0[:, S:]

    def step(t, carry):
        an, ad = carry
        et = emis_ref[t]
        xn = an[:, :, None] + nA
        mn = jnp.max(xn, axis=1)
        cn = mn + jnp.log(jnp.sum(jnp.exp(xn - mn[:, None, :]), axis=1))
        cn = cn + et[:, :S]
        xd = ad[:, :, None] + dA
        md = jnp.max(xd, axis=1)
        cd = md + jnp.log(jnp.sum(jnp.exp(xd - md[:, None, :]), axis=1))
        cd = cd + et[:, S:]
        act = t < seql
        return jnp.where(act, cn, an), jnp.where(act, cd, ad)

    an, ad = jax.lax.fori_loop(1, T, step, (an, ad))

    nf = an + nF_ref[...]
    df = ad + jnp.broadcast_to(dF_ref[...], (B, S))
    mn = jnp.max(nf, axis=1, keepdims=True)
    num = mn + jnp.log(jnp.sum(jnp.exp(nf - mn), axis=1, keepdims=True))
    md = jnp.max(df, axis=1, keepdims=True)
    den = md + jnp.log(jnp.sum(jnp.exp(df - md), axis=1, keepdims=True))
    out_ref[...] = -jnp.sum(num - den, axis=0, keepdims=True)


def _impl(input, seqlengths, num_logA, num_init, num_final, num_state2pdf,
          den_logA, den_init, den_final, den_state2pdf, interpret=False):
    emis = pl.pallas_call(
        _emis_kernel,
        grid=(B,),
        in_specs=[
            pl.BlockSpec((1, T, C), lambda b: (b, 0, 0)),
            pl.BlockSpec((1, 1, S), lambda b: (b, 0, 0)),
            pl.BlockSpec((1, S), lambda b: (0, 0)),
        ],
        out_specs=pl.BlockSpec((1, T, S2), lambda b: (b, 0, 0)),
        out_shape=jax.ShapeDtypeStruct((B, T, S2), jnp.float32),
        interpret=interpret,
    )(input, num_state2pdf.reshape(B, 1, S), den_state2pdf.reshape(1, S))
    emis_t = jnp.transpose(emis, (1, 0, 2))
    loss = pl.pallas_call(
        _fwd_kernel,
        out_shape=jax.ShapeDtypeStruct((1, 1), jnp.float32),
        interpret=interpret,
    )(emis_t, num_logA, den_logA.reshape(1, S, S), num_init,
      den_init.reshape(1, S), num_final, den_final.reshape(1, S),
      seqlengths.reshape(B, 1))
    return loss[0, 0]


def kernel(input, seqlengths, num_logA, num_init, num_final, num_state2pdf,
           den_logA, den_init, den_final, den_state2pdf):
    return _impl(input, seqlengths, num_logA, num_init, num_final,
                 num_state2pdf, den_logA, den_init, den_final, den_state2pdf)
